# Initial kernel scaffold; baseline (speedup 1.0000x reference)
import jax
import jax.numpy as jnp
from jax.experimental import pallas as pl


def kernel(token_ids, weights):
    raise NotImplementedError("write your pallas kernel here")


# speedup vs baseline: 1.4942x; 1.4942x over previous
import functools

import jax
import jax.numpy as jnp
from jax import lax
from jax.experimental import pallas as pl
from jax.experimental.pallas import tpu as pltpu
from jax.experimental.pallas import tpu_sc as plsc

ROWS = 4096
COLS = 200
B = ROWS * COLS
D = 32
NC = 2
NS = 16
NW = NC * NS
PER_W = B // NW
C = 128
NCHUNK = PER_W // C
K = 20
NB = NCHUNK // K
R = K * C

_mesh = plsc.VectorSubcoreMesh(core_axis_name="c", subcore_axis_name="s")


@functools.partial(
    pl.kernel,
    mesh=_mesh,
    out_type=jax.ShapeDtypeStruct((B, D), jnp.float32),
    scratch_types=[
        pltpu.VMEM((NCHUNK, C), jnp.int32),
        pltpu.VMEM((R, D), jnp.float32),
        pltpu.SemaphoreType.DMA,
    ],
    compiler_params=pltpu.CompilerParams(use_tc_tiling_on_sc=False),
)
def _gather(idx_hbm, table_hbm, out_hbm, idx_v, rows_v, sem):
    wid = lax.axis_index("s") * NC + lax.axis_index("c")
    base = wid * PER_W

    pltpu.sync_copy(idx_hbm.at[wid], idx_v)

    def block(b, carry):
        copies = []
        for j in range(K):
            cp = pltpu.make_async_copy(
                table_hbm.at[idx_v.at[b * K + j]],
                rows_v.at[pl.ds(j * C, C)],
                sem,
            )
            cp.start()
            copies.append(cp)
        for cp in copies:
            cp.wait()
        pltpu.sync_copy(rows_v, out_hbm.at[pl.ds(base + b * R, R)])
        return carry

    lax.fori_loop(0, NB, block, 0)


def kernel(token_ids, weights):
    idx = token_ids.reshape(NW, NCHUNK, C)
    out = _gather(idx, weights)
    return out.reshape(ROWS, COLS, D)


# speedup vs baseline: 1.4949x; 1.0005x over previous
import functools

import jax
import jax.numpy as jnp
from jax import lax
from jax.experimental import pallas as pl
from jax.experimental.pallas import tpu as pltpu
from jax.experimental.pallas import tpu_sc as plsc

ROWS = 4096
COLS = 200
B = ROWS * COLS
D = 32
NC = 2
NS = 16
NW = NC * NS
PER_W = B // NW
C = 2560
NCHUNK = PER_W // C
K = 1
NB = NCHUNK // K
R = K * C

_mesh = plsc.VectorSubcoreMesh(core_axis_name="c", subcore_axis_name="s")


@functools.partial(
    pl.kernel,
    mesh=_mesh,
    out_type=jax.ShapeDtypeStruct((B, D), jnp.float32),
    scratch_types=[
        pltpu.VMEM((NCHUNK, C), jnp.int32),
        pltpu.VMEM((R, D), jnp.float32),
        pltpu.SemaphoreType.DMA,
    ],
    compiler_params=pltpu.CompilerParams(use_tc_tiling_on_sc=False),
)
def _gather(idx_hbm, table_hbm, out_hbm, idx_v, rows_v, sem):
    wid = lax.axis_index("s") * NC + lax.axis_index("c")
    base = wid * PER_W

    pltpu.sync_copy(idx_hbm.at[wid], idx_v)

    def block(b, carry):
        copies = []
        for j in range(K):
            cp = pltpu.make_async_copy(
                table_hbm.at[idx_v.at[b * K + j]],
                rows_v.at[pl.ds(j * C, C)],
                sem,
            )
            cp.start()
            copies.append(cp)
        for cp in copies:
            cp.wait()
        pltpu.sync_copy(rows_v, out_hbm.at[pl.ds(base + b * R, R)])
        return carry

    lax.fori_loop(0, NB, block, 0)


def kernel(token_ids, weights):
    idx = token_ids.reshape(NW, NCHUNK, C)
    out = _gather(idx, weights)
    return out.reshape(ROWS, COLS, D)


# speedup vs baseline: 1.5005x; 1.0037x over previous
import functools

import jax
import jax.numpy as jnp
from jax import lax
from jax.experimental import pallas as pl
from jax.experimental.pallas import tpu as pltpu
from jax.experimental.pallas import tpu_sc as plsc

ROWS = 4096
COLS = 200
B = ROWS * COLS
D = 32
NC = 2
NS = 16
NW = NC * NS
PER_W = B // NW
C = 1280
NB = PER_W // C

_mesh = plsc.VectorSubcoreMesh(core_axis_name="c", subcore_axis_name="s")


@functools.partial(
    pl.kernel,
    mesh=_mesh,
    out_type=jax.ShapeDtypeStruct((B, D), jnp.float32),
    scratch_types=[
        pltpu.VMEM((NB, C), jnp.int32),
        pltpu.VMEM((C, D), jnp.float32),
        pltpu.VMEM((C, D), jnp.float32),
        pltpu.SemaphoreType.DMA,
        pltpu.SemaphoreType.DMA,
        pltpu.SemaphoreType.DMA,
        pltpu.SemaphoreType.DMA,
    ],
    compiler_params=pltpu.CompilerParams(use_tc_tiling_on_sc=False),
)
def _gather(idx_hbm, table_hbm, out_hbm, idx_v, rows0, rows1, g0, g1, w0, w1):
    wid = lax.axis_index("s") * NC + lax.axis_index("c")
    base = wid * PER_W

    pltpu.sync_copy(idx_hbm.at[wid], idx_v)

    bufs = (rows0, rows1)
    gsem = (g0, g1)
    wsem = (w0, w1)

    def gcopy(b, p):
        return pltpu.make_async_copy(table_hbm.at[idx_v.at[b]], bufs[p], gsem[p])

    def wcopy(b, p):
        return pltpu.make_async_copy(
            bufs[p], out_hbm.at[pl.ds(base + b * C, C)], wsem[p]
        )

    gcopy(0, 0).start()
    gcopy(1, 1).start()

    def pair(t, carry):
        for p in range(2):
            b = 2 * t + p
            gcopy(b, p).wait()
            wcopy(b, p).start()
            wcopy(b, p).wait()

            @pl.when(b + 2 < NB)
            def _():
                gcopy(b + 2, p).start()

        return carry

    lax.fori_loop(0, NB // 2, pair, 0)


def kernel(token_ids, weights):
    idx = token_ids.reshape(NW, NB, C)
    out = _gather(idx, weights)
    return out.reshape(ROWS, COLS, D)


# speedup vs baseline: 1.5008x; 1.0002x over previous
import functools

import jax
import jax.numpy as jnp
from jax import lax
from jax.experimental import pallas as pl
from jax.experimental.pallas import tpu as pltpu
from jax.experimental.pallas import tpu_sc as plsc

ROWS = 4096
COLS = 200
D = 32
NC = 2
NS = 16
NW = NC * NS
RPW = ROWS // NW
NSLOT = 4

_mesh = plsc.VectorSubcoreMesh(core_axis_name="c", subcore_axis_name="s")


@functools.partial(
    pl.kernel,
    mesh=_mesh,
    out_type=jax.ShapeDtypeStruct((ROWS, COLS, D), jnp.float32),
    scratch_types=[
        pltpu.VMEM((RPW, COLS), jnp.int32),
        [pltpu.VMEM((COLS, D), jnp.float32)] * NSLOT,
        [pltpu.SemaphoreType.DMA] * NSLOT,
        [pltpu.SemaphoreType.DMA] * NSLOT,
    ],
    compiler_params=pltpu.CompilerParams(use_tc_tiling_on_sc=False),
)
def _gather(idx_hbm, table_hbm, out_hbm, idx_v, bufs, gsem, wsem):
    wid = lax.axis_index("s") * NC + lax.axis_index("c")
    base = wid * RPW

    pltpu.sync_copy(idx_hbm.at[pl.ds(base, RPW)], idx_v)

    def gcopy(j, p):
        return pltpu.make_async_copy(table_hbm.at[idx_v.at[j]], bufs[p], gsem[p])

    def wcopy(j, p):
        return pltpu.make_async_copy(bufs[p], out_hbm.at[base + j], wsem[p])

    for p in range(NSLOT):
        gcopy(p, p).start()

    def turn(t, carry):
        for p in range(NSLOT):
            j = NSLOT * t + p
            gcopy(j, p).wait()
            wcopy(j, p).start()
            wcopy(j, p).wait()

            @pl.when(j + NSLOT < RPW)
            def _():
                gcopy(j + NSLOT, p).start()

        return carry

    lax.fori_loop(0, RPW // NSLOT, turn, 0)


def kernel(token_ids, weights):
    return _gather(token_ids, weights)
